# baseline (speedup 1.0000x reference)
import functools

import jax
import jax.numpy as jnp
from jax import lax
from jax.experimental import pallas as pl
from jax.experimental.pallas import tpu as pltpu
from jax.experimental.pallas import tpu_sc as plsc

_NC = 2
_NS = 16
_NW = _NC * _NS


def _zero_vmem_rows(ref, nrows, ncols):
    def body(i, c):
        for j in range(ncols // 16):
            ref[i, pl.ds(j * 16, 16)] = jnp.zeros((16,), jnp.float32)
        return c
    lax.fori_loop(0, nrows, body, 0)


def _zero_and_writeback_slices(N):
    RPS = (N // _NS) // 8 * 8
    TAIL = N - _NS * RPS
    assert TAIL % 8 == 0 or TAIL == 0
    return RPS, TAIL


def _make_sc_degree(N, E):
    EW = E // _NW
    K = 80
    W = 128
    CH = EW // K
    SB = 25
    NB = CH // SB
    RPS, TAIL = _zero_and_writeback_slices(N)
    mesh = plsc.VectorSubcoreMesh(core_axis_name="c", subcore_axis_name="s")

    @functools.partial(
        pl.kernel,
        out_type=jax.ShapeDtypeStruct((_NC, N, W), jnp.float32),
        mesh=mesh,
        scratch_types=[
            pltpu.VMEM((NB, SB, K), jnp.int32),
            pltpu.VMEM((K, W), jnp.float32),
            pltpu.VMEM((K, W), jnp.float32),
            pltpu.VMEM_SHARED((N, W), jnp.float32),
        ],
    )
    def k(dst_hbm, out_hbm, dst_all, ones_v, zero_v, acc_sh):
        cid = lax.axis_index("c")
        sid = lax.axis_index("s")
        wid = cid * _NS + sid
        pltpu.sync_copy(dst_hbm.at[wid], dst_all)
        def fill(i, c):
            for j in range(W // 16):
                ones_v[i, pl.ds(j * 16, 16)] = jnp.ones((16,), jnp.float32)
            return c
        lax.fori_loop(0, K, fill, 0)
        _zero_vmem_rows(zero_v, K, W)
        base_r = sid * RPS
        nfull = RPS // K
        rem = RPS - nfull * K
        def zcp(i, c):
            pltpu.sync_copy(zero_v, acc_sh.at[pl.ds(base_r + i * K, K)])
            return c
        lax.fori_loop(0, nfull, zcp, 0)
        if rem:
            pltpu.sync_copy(zero_v.at[pl.ds(0, rem)],
                            acc_sh.at[pl.ds(base_r + nfull * K, rem)])
        if TAIL:
            @pl.when(sid == _NS - 1)
            def _():
                pltpu.sync_copy(zero_v.at[pl.ds(0, TAIL)],
                                acc_sh.at[pl.ds(_NS * RPS, TAIL)])
        plsc.subcore_barrier()
        for t in range(NB):
            def body(i, c):
                pltpu.sync_copy(ones_v, acc_sh.at[dst_all.at[t, i]], add=True)
                return c
            lax.fori_loop(0, SB, body, 0)
        plsc.subcore_barrier()
        pltpu.sync_copy(acc_sh.at[pl.ds(base_r, RPS)],
                        out_hbm.at[cid].at[pl.ds(base_r, RPS)])
        if TAIL:
            @pl.when(sid == _NS - 1)
            def _():
                pltpu.sync_copy(acc_sh.at[pl.ds(_NS * RPS, TAIL)],
                                out_hbm.at[cid].at[pl.ds(_NS * RPS, TAIL)])

    return k


def _make_sc_scatter(N, D, E):
    EW = E // _NW
    K = 80
    CH = EW // K
    RPS, TAIL = _zero_and_writeback_slices(N)
    mesh = plsc.VectorSubcoreMesh(core_axis_name="c", subcore_axis_name="s")

    SB = 25
    NB = CH // SB
    assert CH == NB * SB and SB % 2 == 1

    @functools.partial(
        pl.kernel,
        out_type=jax.ShapeDtypeStruct((_NC, N, D), jnp.float32),
        mesh=mesh,
        scratch_types=[
            pltpu.VMEM((SB, K), jnp.int32),
            pltpu.VMEM((SB, K), jnp.int32),
            pltpu.VMEM((SB, K), jnp.int32),
            pltpu.VMEM((SB, K), jnp.int32),
            pltpu.VMEM((K, D), jnp.float32),
            pltpu.VMEM((K, D), jnp.float32),
            pltpu.VMEM_SHARED((N, D), jnp.float32),
            pltpu.SemaphoreType.DMA,
            pltpu.SemaphoreType.DMA,
            pltpu.SemaphoreType.DMA,
            pltpu.SemaphoreType.DMA,
            pltpu.SemaphoreType.DMA,
        ],
    )
    def k(mp_hbm, src_hbm, dst_hbm, out_hbm,
          src_0, dst_0, src_1, dst_1, rows_a, rows_b, acc_sh,
          sem_i, sem_a, sem_a2, sem_b, sem_b2):
        cid = lax.axis_index("c")
        sid = lax.axis_index("s")
        wid = cid * _NS + sid
        pltpu.async_copy(src_hbm.at[wid, 0], src_0, sem_i)
        pltpu.async_copy(dst_hbm.at[wid, 0], dst_0, sem_i)
        _zero_vmem_rows(rows_a, K, D)
        base_r = sid * RPS
        nfull = RPS // K
        rem = RPS - nfull * K
        def zcp(i, c):
            pltpu.sync_copy(rows_a, acc_sh.at[pl.ds(base_r + i * K, K)])
            return c
        lax.fori_loop(0, nfull, zcp, 0)
        if rem:
            pltpu.sync_copy(rows_a.at[pl.ds(0, rem)],
                            acc_sh.at[pl.ds(base_r + nfull * K, rem)])
        if TAIL:
            @pl.when(sid == _NS - 1)
            def _():
                pltpu.sync_copy(rows_a.at[pl.ds(0, TAIL)],
                                acc_sh.at[pl.ds(_NS * RPS, TAIL)])
        plsc.subcore_barrier()

        H = K // 2

        def fire(sv, j, r_v, sem, sem2):
            pltpu.async_copy(mp_hbm.at[sv.at[j, pl.ds(0, H)]],
                             r_v.at[pl.ds(0, H)], sem)
            pltpu.async_copy(mp_hbm.at[sv.at[j, pl.ds(H, H)]],
                             r_v.at[pl.ds(H, H)], sem2)

        def drain_and_add(sv, dv, j, r_v, sem, sem2):
            pltpu.make_async_copy(mp_hbm.at[sv.at[j, pl.ds(0, H)]],
                                  r_v.at[pl.ds(0, H)], sem).wait()
            pltpu.make_async_copy(mp_hbm.at[sv.at[j, pl.ds(H, H)]],
                                  r_v.at[pl.ds(H, H)], sem2).wait()
            pltpu.sync_copy(r_v, acc_sh.at[dv.at[j]], add=True)

        for t in range(NB):
            sv, dv = (src_0, dst_0) if t % 2 == 0 else (src_1, dst_1)
            nsv, ndv = (src_1, dst_1) if t % 2 == 0 else (src_0, dst_0)
            pltpu.make_async_copy(src_hbm.at[wid, t], sv, sem_i).wait()
            pltpu.make_async_copy(dst_hbm.at[wid, t], dv, sem_i).wait()
            if t + 1 < NB:
                pltpu.async_copy(src_hbm.at[wid, t + 1], nsv, sem_i)
                pltpu.async_copy(dst_hbm.at[wid, t + 1], ndv, sem_i)
            fire(sv, 0, rows_a, sem_a, sem_a2)
            def pair(j, c):
                fire(sv, 2 * j + 1, rows_b, sem_b, sem_b2)
                drain_and_add(sv, dv, 2 * j, rows_a, sem_a, sem_a2)
                fire(sv, 2 * j + 2, rows_a, sem_a, sem_a2)
                drain_and_add(sv, dv, 2 * j + 1, rows_b, sem_b, sem_b2)
                return c
            lax.fori_loop(0, (SB - 1) // 2, pair, 0)
            drain_and_add(sv, dv, SB - 1, rows_a, sem_a, sem_a2)
        plsc.subcore_barrier()
        pltpu.sync_copy(acc_sh.at[pl.ds(base_r, RPS)],
                        out_hbm.at[cid].at[pl.ds(base_r, RPS)])
        if TAIL:
            @pl.when(sid == _NS - 1)
            def _():
                pltpu.sync_copy(acc_sh.at[pl.ds(_NS * RPS, TAIL)],
                                out_hbm.at[cid].at[pl.ds(_NS * RPS, TAIL)])

    return k


def _tc_embed_body(degp_ref, x_ref, wemb_ref, bemb_ref, w0_ref,
                   h0_ref, mp_ref, dis_ref):
    deg = 1.0 + degp_ref[0, :, 0:1] + degp_ref[1, :, 0:1]
    dis = lax.rsqrt(deg)
    dis_ref[...] = dis
    h0 = jnp.dot(x_ref[...], wemb_ref[...],
                 preferred_element_type=jnp.float32) + bemb_ref[...]
    h0_ref[...] = h0
    mp_ref[...] = dis * jnp.dot(h0, w0_ref[...],
                                preferred_element_type=jnp.float32)


def _bn_relu_res(S_ref, mp_ref, h_ref, dis_ref, b_ref, g_ref, be_ref):
    dis = dis_ref[...]
    mp = mp_ref[...]
    agg = dis * (S_ref[0] + S_ref[1] + mp) + b_ref[...]
    mu = jnp.mean(agg, axis=0, keepdims=True)
    var = jnp.mean((agg - mu) * (agg - mu), axis=0, keepdims=True)
    hbn = (agg - mu) * lax.rsqrt(var + 1e-5) * g_ref[...] + be_ref[...]
    return h_ref[...] + jnp.maximum(hbn, 0.0)


def _tc_layer_body(S_ref, mp_ref, h_ref, dis_ref, b_ref, g_ref, be_ref,
                   wn_ref, hn_ref, mpn_ref):
    hn = _bn_relu_res(S_ref, mp_ref, h_ref, dis_ref, b_ref, g_ref, be_ref)
    hn_ref[...] = hn
    mpn_ref[...] = dis_ref[...] * jnp.dot(hn, wn_ref[...],
                                          preferred_element_type=jnp.float32)


def _tc_final_body(S_ref, mp_ref, h_ref, dis_ref, b_ref, g_ref, be_ref,
                   batchT_ref, hg_ref, *, nbatch):
    hn = _bn_relu_res(S_ref, mp_ref, h_ref, dis_ref, b_ref, g_ref, be_ref)
    n = hn.shape[0]
    oneh = (batchT_ref[...] ==
            lax.broadcasted_iota(jnp.int32, (nbatch, n), 0)
            ).astype(jnp.float32)
    counts = jnp.sum(oneh, axis=1, keepdims=True)
    hg = jnp.dot(oneh, hn, preferred_element_type=jnp.float32)
    hg_ref[...] = hg / jnp.maximum(counts, 1.0)


def kernel(x, W_emb, b_emb, W, b, gamma, beta, edge_index, batch):
    N, D = x.shape
    E = edge_index.shape[1]
    L = W.shape[0]
    B = 64
    f32 = jnp.float32

    EW = E // _NW
    K, SB = 80, 25
    NB = EW // K // SB
    src = edge_index[0].reshape(_NW, NB, SB, K)
    dst = edge_index[1].reshape(_NW, NB, SB, K)
    batchT = batch.reshape(1, N)

    deg_part = _make_sc_degree(N, E)(dst)

    h, mp, dis = pl.pallas_call(
        _tc_embed_body,
        out_shape=(
            jax.ShapeDtypeStruct((N, D), f32),
            jax.ShapeDtypeStruct((N, D), f32),
            jax.ShapeDtypeStruct((N, 1), f32),
        ),
    )(deg_part, x, W_emb, b_emb.reshape(1, D), W[0])

    sc_scatter = _make_sc_scatter(N, D, E)

    for l in range(L):
        S = sc_scatter(mp, src, dst)
        if l + 1 < L:
            h, mp = pl.pallas_call(
                _tc_layer_body,
                out_shape=(
                    jax.ShapeDtypeStruct((N, D), f32),
                    jax.ShapeDtypeStruct((N, D), f32),
                ),
            )(S, mp, h, dis, b[l].reshape(1, D), gamma[l].reshape(1, D),
              beta[l].reshape(1, D), W[l + 1])
        else:
            hg = pl.pallas_call(
                functools.partial(_tc_final_body, nbatch=B),
                out_shape=jax.ShapeDtypeStruct((B, D), f32),
            )(S, mp, h, dis, b[l].reshape(1, D), gamma[l].reshape(1, D),
              beta[l].reshape(1, D), batchT)
    return hg


# speedup vs baseline: 1.0014x; 1.0014x over previous
import functools

import jax
import jax.numpy as jnp
from jax import lax
from jax.experimental import pallas as pl
from jax.experimental.pallas import tpu as pltpu
from jax.experimental.pallas import tpu_sc as plsc

_NC = 2
_NS = 16
_NW = _NC * _NS


def _zero_vmem_rows(ref, nrows, ncols):
    def body(i, c):
        for j in range(ncols // 16):
            ref[i, pl.ds(j * 16, 16)] = jnp.zeros((16,), jnp.float32)
        return c
    lax.fori_loop(0, nrows, body, 0)


def _zero_and_writeback_slices(N):
    RPS = (N // _NS) // 8 * 8
    TAIL = N - _NS * RPS
    assert TAIL % 8 == 0 or TAIL == 0
    return RPS, TAIL


def _make_sc_degree(N, E, W=128):
    EW = E // _NW
    K = 80
    CH = EW // K
    SB = 25
    NB = CH // SB
    RPS, TAIL = _zero_and_writeback_slices(N)
    mesh = plsc.VectorSubcoreMesh(core_axis_name="c", subcore_axis_name="s")

    @functools.partial(
        pl.kernel,
        out_type=jax.ShapeDtypeStruct((_NC, N, W), jnp.float32),
        mesh=mesh,
        scratch_types=[
            pltpu.VMEM((CH, K), jnp.int32),
            pltpu.VMEM((K, W), jnp.float32),
            pltpu.VMEM((K, W), jnp.float32),
            pltpu.VMEM_SHARED((N, W), jnp.float32),
            pltpu.SemaphoreType.DMA,
        ],
    )
    def k(dst_hbm, out_hbm, dst_all, ones_v, zero_v, acc_sh, sem):
        cid = lax.axis_index("c")
        sid = lax.axis_index("s")
        wid = cid * _NS + sid
        pltpu.sync_copy(dst_hbm.at[wid], dst_all)
        def fill(i, c):
            for j in range(W // 16):
                ones_v[i, pl.ds(j * 16, 16)] = jnp.ones((16,), jnp.float32)
            return c
        lax.fori_loop(0, K, fill, 0)
        _zero_vmem_rows(zero_v, K, W)
        base_r = sid * RPS
        nfull = RPS // K
        rem = RPS - nfull * K
        def zcp(i, c):
            pltpu.sync_copy(zero_v, acc_sh.at[pl.ds(base_r + i * K, K)])
            return c
        lax.fori_loop(0, nfull, zcp, 0)
        if rem:
            pltpu.sync_copy(zero_v.at[pl.ds(0, rem)],
                            acc_sh.at[pl.ds(base_r + nfull * K, rem)])
        if TAIL:
            @pl.when(sid == _NS - 1)
            def _():
                pltpu.sync_copy(zero_v.at[pl.ds(0, TAIL)],
                                acc_sh.at[pl.ds(_NS * RPS, TAIL)])
        plsc.subcore_barrier()
        WIN = 8
        for i in range(WIN):
            pltpu.async_copy(ones_v, acc_sh.at[dst_all.at[i]], sem, add=True)
        def body(f, c):
            pltpu.async_copy(ones_v, acc_sh.at[dst_all.at[f + WIN]], sem,
                             add=True)
            pltpu.make_async_copy(ones_v, acc_sh.at[dst_all.at[f]], sem).wait()
            return c
        lax.fori_loop(0, CH - WIN, body, 0)
        for i in range(WIN):
            pltpu.make_async_copy(ones_v, acc_sh.at[dst_all.at[0]], sem).wait()
        plsc.subcore_barrier()
        pltpu.sync_copy(acc_sh.at[pl.ds(base_r, RPS)],
                        out_hbm.at[cid].at[pl.ds(base_r, RPS)])
        if TAIL:
            @pl.when(sid == _NS - 1)
            def _():
                pltpu.sync_copy(acc_sh.at[pl.ds(_NS * RPS, TAIL)],
                                out_hbm.at[cid].at[pl.ds(_NS * RPS, TAIL)])

    return k


def _make_sc_scatter(N, D, E):
    EW = E // _NW
    K = 80
    CH = EW // K
    RPS, TAIL = _zero_and_writeback_slices(N)
    mesh = plsc.VectorSubcoreMesh(core_axis_name="c", subcore_axis_name="s")

    SB = 25
    NB = CH // SB
    assert CH == NB * SB and SB % 2 == 1

    @functools.partial(
        pl.kernel,
        out_type=jax.ShapeDtypeStruct((_NC, N, D), jnp.float32),
        mesh=mesh,
        scratch_types=[
            pltpu.VMEM((SB, K), jnp.int32),
            pltpu.VMEM((SB, K), jnp.int32),
            pltpu.VMEM((SB, K), jnp.int32),
            pltpu.VMEM((SB, K), jnp.int32),
            pltpu.VMEM((K, D), jnp.float32),
            pltpu.VMEM((K, D), jnp.float32),
            pltpu.VMEM_SHARED((N, D), jnp.float32),
            pltpu.SemaphoreType.DMA,
            pltpu.SemaphoreType.DMA,
            pltpu.SemaphoreType.DMA,
            pltpu.SemaphoreType.DMA,
            pltpu.SemaphoreType.DMA,
        ],
    )
    def k(mp_hbm, src_hbm, dst_hbm, out_hbm,
          src_0, dst_0, src_1, dst_1, rows_a, rows_b, acc_sh,
          sem_i, sem_a, sem_a2, sem_b, sem_b2):
        cid = lax.axis_index("c")
        sid = lax.axis_index("s")
        wid = cid * _NS + sid
        pltpu.async_copy(src_hbm.at[wid, 0], src_0, sem_i)
        pltpu.async_copy(dst_hbm.at[wid, 0], dst_0, sem_i)
        _zero_vmem_rows(rows_a, K, D)
        base_r = sid * RPS
        nfull = RPS // K
        rem = RPS - nfull * K
        def zcp(i, c):
            pltpu.sync_copy(rows_a, acc_sh.at[pl.ds(base_r + i * K, K)])
            return c
        lax.fori_loop(0, nfull, zcp, 0)
        if rem:
            pltpu.sync_copy(rows_a.at[pl.ds(0, rem)],
                            acc_sh.at[pl.ds(base_r + nfull * K, rem)])
        if TAIL:
            @pl.when(sid == _NS - 1)
            def _():
                pltpu.sync_copy(rows_a.at[pl.ds(0, TAIL)],
                                acc_sh.at[pl.ds(_NS * RPS, TAIL)])
        plsc.subcore_barrier()

        H = K // 2

        def fire(sv, j, r_v, sem, sem2):
            pltpu.async_copy(mp_hbm.at[sv.at[j, pl.ds(0, H)]],
                             r_v.at[pl.ds(0, H)], sem)
            pltpu.async_copy(mp_hbm.at[sv.at[j, pl.ds(H, H)]],
                             r_v.at[pl.ds(H, H)], sem2)

        def drain_and_add(sv, dv, j, r_v, sem, sem2):
            pltpu.make_async_copy(mp_hbm.at[sv.at[j, pl.ds(0, H)]],
                                  r_v.at[pl.ds(0, H)], sem).wait()
            pltpu.make_async_copy(mp_hbm.at[sv.at[j, pl.ds(H, H)]],
                                  r_v.at[pl.ds(H, H)], sem2).wait()
            pltpu.sync_copy(r_v, acc_sh.at[dv.at[j]], add=True)

        for t in range(NB):
            sv, dv = (src_0, dst_0) if t % 2 == 0 else (src_1, dst_1)
            nsv, ndv = (src_1, dst_1) if t % 2 == 0 else (src_0, dst_0)
            pltpu.make_async_copy(src_hbm.at[wid, t], sv, sem_i).wait()
            pltpu.make_async_copy(dst_hbm.at[wid, t], dv, sem_i).wait()
            if t + 1 < NB:
                pltpu.async_copy(src_hbm.at[wid, t + 1], nsv, sem_i)
                pltpu.async_copy(dst_hbm.at[wid, t + 1], ndv, sem_i)
            fire(sv, 0, rows_a, sem_a, sem_a2)
            def pair(j, c):
                fire(sv, 2 * j + 1, rows_b, sem_b, sem_b2)
                drain_and_add(sv, dv, 2 * j, rows_a, sem_a, sem_a2)
                fire(sv, 2 * j + 2, rows_a, sem_a, sem_a2)
                drain_and_add(sv, dv, 2 * j + 1, rows_b, sem_b, sem_b2)
                return c
            lax.fori_loop(0, (SB - 1) // 2, pair, 0)
            drain_and_add(sv, dv, SB - 1, rows_a, sem_a, sem_a2)
        plsc.subcore_barrier()
        pltpu.sync_copy(acc_sh.at[pl.ds(base_r, RPS)],
                        out_hbm.at[cid].at[pl.ds(base_r, RPS)])
        if TAIL:
            @pl.when(sid == _NS - 1)
            def _():
                pltpu.sync_copy(acc_sh.at[pl.ds(_NS * RPS, TAIL)],
                                out_hbm.at[cid].at[pl.ds(_NS * RPS, TAIL)])

    return k


def _tc_embed_body(degp_ref, x_ref, wemb_ref, bemb_ref, w0_ref,
                   h0_ref, mp_ref, dis_ref):
    deg = 1.0 + degp_ref[0, :, 0:1] + degp_ref[1, :, 0:1]
    dis = lax.rsqrt(deg)
    dis_ref[...] = dis
    h0 = jnp.dot(x_ref[...], wemb_ref[...],
                 preferred_element_type=jnp.float32) + bemb_ref[...]
    h0_ref[...] = h0
    mp_ref[...] = dis * jnp.dot(h0, w0_ref[...],
                                preferred_element_type=jnp.float32)


def _bn_relu_res(S_ref, mp_ref, h_ref, dis_ref, b_ref, g_ref, be_ref):
    dis = dis_ref[...]
    mp = mp_ref[...]
    agg = dis * (S_ref[0] + S_ref[1] + mp) + b_ref[...]
    mu = jnp.mean(agg, axis=0, keepdims=True)
    var = jnp.mean((agg - mu) * (agg - mu), axis=0, keepdims=True)
    hbn = (agg - mu) * lax.rsqrt(var + 1e-5) * g_ref[...] + be_ref[...]
    return h_ref[...] + jnp.maximum(hbn, 0.0)


def _tc_layer_body(S_ref, mp_ref, h_ref, dis_ref, b_ref, g_ref, be_ref,
                   wn_ref, hn_ref, mpn_ref):
    hn = _bn_relu_res(S_ref, mp_ref, h_ref, dis_ref, b_ref, g_ref, be_ref)
    hn_ref[...] = hn
    mpn_ref[...] = dis_ref[...] * jnp.dot(hn, wn_ref[...],
                                          preferred_element_type=jnp.float32)


def _tc_final_body(S_ref, mp_ref, h_ref, dis_ref, b_ref, g_ref, be_ref,
                   batchT_ref, hg_ref, *, nbatch):
    hn = _bn_relu_res(S_ref, mp_ref, h_ref, dis_ref, b_ref, g_ref, be_ref)
    n = hn.shape[0]
    oneh = (batchT_ref[...] ==
            lax.broadcasted_iota(jnp.int32, (nbatch, n), 0)
            ).astype(jnp.float32)
    counts = jnp.sum(oneh, axis=1, keepdims=True)
    hg = jnp.dot(oneh, hn, preferred_element_type=jnp.float32)
    hg_ref[...] = hg / jnp.maximum(counts, 1.0)


def kernel(x, W_emb, b_emb, W, b, gamma, beta, edge_index, batch):
    N, D = x.shape
    E = edge_index.shape[1]
    L = W.shape[0]
    B = 64
    f32 = jnp.float32

    EW = E // _NW
    K, SB = 80, 25
    NB = EW // K // SB
    src = edge_index[0].reshape(_NW, NB, SB, K)
    dst = edge_index[1].reshape(_NW, NB, SB, K)
    batchT = batch.reshape(1, N)

    deg_part = _make_sc_degree(N, E)(edge_index[1].reshape(_NW, EW // K, K))

    h, mp, dis = pl.pallas_call(
        _tc_embed_body,
        out_shape=(
            jax.ShapeDtypeStruct((N, D), f32),
            jax.ShapeDtypeStruct((N, D), f32),
            jax.ShapeDtypeStruct((N, 1), f32),
        ),
    )(deg_part, x, W_emb, b_emb.reshape(1, D), W[0])

    sc_scatter = _make_sc_scatter(N, D, E)

    for l in range(L):
        S = sc_scatter(mp, src, dst)
        if l + 1 < L:
            h, mp = pl.pallas_call(
                _tc_layer_body,
                out_shape=(
                    jax.ShapeDtypeStruct((N, D), f32),
                    jax.ShapeDtypeStruct((N, D), f32),
                ),
            )(S, mp, h, dis, b[l].reshape(1, D), gamma[l].reshape(1, D),
              beta[l].reshape(1, D), W[l + 1])
        else:
            hg = pl.pallas_call(
                functools.partial(_tc_final_body, nbatch=B),
                out_shape=jax.ShapeDtypeStruct((B, D), f32),
            )(S, mp, h, dis, b[l].reshape(1, D), gamma[l].reshape(1, D),
              beta[l].reshape(1, D), batchT)
    return hg
